# Initial kernel scaffold; baseline (speedup 1.0000x reference)
import jax
import jax.numpy as jnp
from jax.experimental import pallas as pl


def kernel(indices, tables, W1, b1, W2, b2):
    raise NotImplementedError("write your pallas kernel here")


# speedup vs baseline: 8.0970x; 8.0970x over previous
import functools

import jax
import jax.numpy as jnp
from jax import lax
from jax.experimental import pallas as pl
from jax.experimental.pallas import tpu as pltpu
from jax.experimental.pallas import tpu_sc as plsc

CHUNK = 128
GROUP = 8


def _sc_gather(flat_idx, flat_tables, nw, rpw, d):
    nch = rpw // CHUNK
    ngroups = nch // GROUP
    grows = GROUP * CHUNK
    mesh = plsc.VectorSubcoreMesh(core_axis_name="c", subcore_axis_name="s")
    nc = 2

    @functools.partial(
        pl.kernel,
        mesh=mesh,
        compiler_params=pltpu.CompilerParams(use_tc_tiling_on_sc=False),
        out_type=jax.ShapeDtypeStruct((nw, rpw, d), jnp.float32),
        scratch_types=[
            pltpu.VMEM((nch, CHUNK), jnp.int32),
            pltpu.VMEM((grows, d), jnp.float32),
            pltpu.SemaphoreType.DMA,
        ],
    )
    def k(idx_hbm, tab_hbm, out_hbm, idx_v, rows_v, sem):
        wid = lax.axis_index("s") * nc + lax.axis_index("c")
        pltpu.sync_copy(idx_hbm.at[wid], idx_v)

        def body(g, carry):
            handles = []
            for j in range(GROUP):
                h = pltpu.async_copy(
                    tab_hbm.at[idx_v.at[g * GROUP + j]],
                    rows_v.at[pl.ds(j * CHUNK, CHUNK)],
                    sem,
                )
                handles.append(h)
            for h in handles:
                h.wait()
            pltpu.sync_copy(rows_v, out_hbm.at[wid, pl.ds(g * grows, grows)])
            return carry

        lax.fori_loop(0, ngroups, body, 0)

    return k(flat_idx, flat_tables)


def _tc_mlp(combined, W1, b1, W2, b2, tb=1024):
    bsz, fd = combined.shape
    h1 = W1.shape[1]
    ed = W2.shape[1]

    def body(x_ref, w1_ref, b1_ref, w2_ref, b2_ref, o_ref):
        h = jnp.dot(x_ref[...], w1_ref[...], preferred_element_type=jnp.float32)
        h = jnp.maximum(h + b1_ref[...], 0.0)
        o = jnp.dot(h, w2_ref[...], preferred_element_type=jnp.float32)
        o_ref[...] = jnp.maximum(o + b2_ref[...], 0.0)

    return pl.pallas_call(
        body,
        grid=(bsz // tb,),
        in_specs=[
            pl.BlockSpec((tb, fd), lambda i: (i, 0)),
            pl.BlockSpec((fd, h1), lambda i: (0, 0)),
            pl.BlockSpec((1, h1), lambda i: (0, 0)),
            pl.BlockSpec((h1, ed), lambda i: (0, 0)),
            pl.BlockSpec((1, ed), lambda i: (0, 0)),
        ],
        out_specs=pl.BlockSpec((tb, ed), lambda i: (i, 0)),
        out_shape=jax.ShapeDtypeStruct((bsz, ed), jnp.float32),
    )(combined, W1, b1.reshape(1, h1), W2, b2.reshape(1, ed))


def kernel(indices, tables, W1, b1, W2, b2):
    f, b = indices.shape
    _, v, d = tables.shape
    nw = 32
    total = b * f
    rpw = total // nw

    offsets = (jnp.arange(f, dtype=jnp.int32) * v)[None, :]
    flat_idx = (indices.astype(jnp.int32).T + offsets).reshape(nw, rpw // CHUNK, CHUNK)
    flat_tables = tables.reshape(f * v, d)

    rows = _sc_gather(flat_idx, flat_tables, nw, rpw, d)
    combined = rows.reshape(b, f * d)
    return _tc_mlp(combined, W1, b1, W2, b2)


# speedup vs baseline: 12.7321x; 1.5724x over previous
import functools

import jax
import jax.numpy as jnp
from jax import lax
from jax.experimental import pallas as pl
from jax.experimental.pallas import tpu as pltpu
from jax.experimental.pallas import tpu_sc as plsc

CHUNK = 128
GROUP = 8
VCHUNK = 12800


def _tc_repack(t_dv, f, v, d):
    lanes = 128
    fold = lanes // d
    ng = (f + fold - 1) // fold
    nv = (v + VCHUNK - 1) // VCHUNK

    def body(x_ref, o_ref):
        parts = [jnp.transpose(x_ref[k], (1, 0)) for k in range(fold)]
        o_ref[0] = jnp.concatenate(parts, axis=1)

    return pl.pallas_call(
        body,
        grid=(ng, nv),
        in_specs=[pl.BlockSpec((fold, d, VCHUNK), lambda i, j: (i, 0, j))],
        out_specs=pl.BlockSpec((1, VCHUNK, lanes), lambda i, j: (i, j, 0)),
        out_shape=jax.ShapeDtypeStruct((ng, v, lanes), jnp.float32),
    )(t_dv)


def _sc_gather(flat_idx, flat_tables, nw, rpw, d):
    nch = rpw // CHUNK
    ngroups = nch // GROUP
    grows = GROUP * CHUNK
    mesh = plsc.VectorSubcoreMesh(core_axis_name="c", subcore_axis_name="s")
    nc = 2

    @functools.partial(
        pl.kernel,
        mesh=mesh,
        compiler_params=pltpu.CompilerParams(use_tc_tiling_on_sc=False),
        out_type=jax.ShapeDtypeStruct((nw, rpw, d), jnp.float32),
        scratch_types=[
            pltpu.VMEM((nch, CHUNK), jnp.int32),
            pltpu.VMEM((grows, d), jnp.float32),
            pltpu.SemaphoreType.DMA,
        ],
    )
    def k(idx_hbm, tab_hbm, out_hbm, idx_v, rows_v, sem):
        wid = lax.axis_index("s") * nc + lax.axis_index("c")
        pltpu.sync_copy(idx_hbm.at[wid], idx_v)

        def body(g, carry):
            handles = []
            for j in range(GROUP):
                h = pltpu.async_copy(
                    tab_hbm.at[idx_v.at[g * GROUP + j]],
                    rows_v.at[pl.ds(j * CHUNK, CHUNK)],
                    sem,
                )
                handles.append(h)
            for h in handles:
                h.wait()
            pltpu.sync_copy(rows_v, out_hbm.at[wid, pl.ds(g * grows, grows)])
            return carry

        lax.fori_loop(0, ngroups, body, 0)

    return k(flat_idx, flat_tables)


def _tc_mlp(combined, W1, b1, W2, b2, tb=1024):
    bsz, fd = combined.shape
    h1 = W1.shape[1]
    ed = W2.shape[1]

    def body(x_ref, w1_ref, b1_ref, w2_ref, b2_ref, o_ref):
        h = jnp.dot(x_ref[...], w1_ref[...], preferred_element_type=jnp.float32)
        h = jnp.maximum(h + b1_ref[...], 0.0)
        o = jnp.dot(h, w2_ref[...], preferred_element_type=jnp.float32)
        o_ref[...] = jnp.maximum(o + b2_ref[...], 0.0)

    return pl.pallas_call(
        body,
        grid=(bsz // tb,),
        in_specs=[
            pl.BlockSpec((tb, fd), lambda i: (i, 0)),
            pl.BlockSpec((fd, h1), lambda i: (0, 0)),
            pl.BlockSpec((1, h1), lambda i: (0, 0)),
            pl.BlockSpec((h1, ed), lambda i: (0, 0)),
            pl.BlockSpec((1, ed), lambda i: (0, 0)),
        ],
        out_specs=pl.BlockSpec((tb, ed), lambda i: (i, 0)),
        out_shape=jax.ShapeDtypeStruct((bsz, ed), jnp.float32),
    )(combined, W1, b1.reshape(1, h1), W2, b2.reshape(1, ed))


def kernel(indices, tables, W1, b1, W2, b2):
    f, b = indices.shape
    _, v, d = tables.shape
    nw = 32
    total = b * f
    rpw = total // nw

    fold = 128 // d
    farange = jnp.arange(f, dtype=jnp.int32)
    offsets = ((farange // fold) * (v * fold) + (farange % fold))[None, :]
    flat_idx = (indices.astype(jnp.int32).T * fold + offsets).reshape(
        nw, rpw // CHUNK, CHUNK
    )
    packed = _tc_repack(jnp.transpose(tables, (0, 2, 1)), f, v, d)
    flat_tables = packed.reshape(packed.shape[0] * v * fold, d)

    rows = _sc_gather(flat_idx, flat_tables, nw, rpw, d)
    combined = rows.reshape(b, f * d)
    return _tc_mlp(combined, W1, b1, W2, b2)


# speedup vs baseline: 18.8262x; 1.4786x over previous
import functools

import jax
import jax.numpy as jnp
from jax import lax
from jax.experimental import pallas as pl
from jax.experimental.pallas import tpu as pltpu
from jax.experimental.pallas import tpu_sc as plsc

CHUNK = 128
GROUP = 8
VCHUNK = 12800


def _tc_repack(t_dv, f, v, d):
    lanes = 128
    fold = lanes // d
    ng = (f + fold - 1) // fold
    nv = (v + VCHUNK - 1) // VCHUNK

    def body(x_ref, o_ref):
        acc = None
        for k in range(fold):
            e_k = (
                lax.broadcasted_iota(jnp.int32, (d, lanes), 0) + k * d
                == lax.broadcasted_iota(jnp.int32, (d, lanes), 1)
            ).astype(jnp.float32)
            p = lax.dot_general(
                x_ref[k],
                e_k,
                (((0,), (0,)), ((), ())),
                preferred_element_type=jnp.float32,
            )
            acc = p if acc is None else acc + p
        o_ref[0] = acc

    return pl.pallas_call(
        body,
        grid=(ng, nv),
        in_specs=[pl.BlockSpec((fold, d, VCHUNK), lambda i, j: (i, 0, j))],
        out_specs=pl.BlockSpec((1, VCHUNK, lanes), lambda i, j: (i, j, 0)),
        out_shape=jax.ShapeDtypeStruct((ng, v, lanes), jnp.float32),
    )(t_dv)


def _sc_gather(flat_idx, flat_tables, nw, rpw, d):
    nch = rpw // CHUNK
    ngroups = nch // GROUP
    grows = GROUP * CHUNK
    mesh = plsc.VectorSubcoreMesh(core_axis_name="c", subcore_axis_name="s")
    nc = 2

    @functools.partial(
        pl.kernel,
        mesh=mesh,
        compiler_params=pltpu.CompilerParams(use_tc_tiling_on_sc=False),
        out_type=jax.ShapeDtypeStruct((nw, rpw, d), jnp.float32),
        scratch_types=[
            pltpu.VMEM((nch, CHUNK), jnp.int32),
            pltpu.VMEM((grows, d), jnp.float32),
            pltpu.SemaphoreType.DMA,
        ],
    )
    def k(idx_hbm, tab_hbm, out_hbm, idx_v, rows_v, sem):
        wid = lax.axis_index("s") * nc + lax.axis_index("c")
        pltpu.sync_copy(idx_hbm.at[wid], idx_v)

        def body(g, carry):
            handles = []
            for j in range(GROUP):
                h = pltpu.async_copy(
                    tab_hbm.at[idx_v.at[g * GROUP + j]],
                    rows_v.at[pl.ds(j * CHUNK, CHUNK)],
                    sem,
                )
                handles.append(h)
            for h in handles:
                h.wait()
            pltpu.sync_copy(rows_v, out_hbm.at[wid, pl.ds(g * grows, grows)])
            return carry

        lax.fori_loop(0, ngroups, body, 0)

    return k(flat_idx, flat_tables)


def _tc_mlp(combined, W1, b1, W2, b2, tb=1024):
    bsz, fd = combined.shape
    h1 = W1.shape[1]
    ed = W2.shape[1]

    def body(x_ref, w1_ref, b1_ref, w2_ref, b2_ref, o_ref):
        h = jnp.dot(x_ref[...], w1_ref[...], preferred_element_type=jnp.float32)
        h = jnp.maximum(h + b1_ref[...], 0.0)
        o = jnp.dot(h, w2_ref[...], preferred_element_type=jnp.float32)
        o_ref[...] = jnp.maximum(o + b2_ref[...], 0.0)

    return pl.pallas_call(
        body,
        grid=(bsz // tb,),
        in_specs=[
            pl.BlockSpec((tb, fd), lambda i: (i, 0)),
            pl.BlockSpec((fd, h1), lambda i: (0, 0)),
            pl.BlockSpec((1, h1), lambda i: (0, 0)),
            pl.BlockSpec((h1, ed), lambda i: (0, 0)),
            pl.BlockSpec((1, ed), lambda i: (0, 0)),
        ],
        out_specs=pl.BlockSpec((tb, ed), lambda i: (i, 0)),
        out_shape=jax.ShapeDtypeStruct((bsz, ed), jnp.float32),
    )(combined, W1, b1.reshape(1, h1), W2, b2.reshape(1, ed))


def kernel(indices, tables, W1, b1, W2, b2):
    f, b = indices.shape
    _, v, d = tables.shape
    nw = 32
    total = b * f
    rpw = total // nw

    fold = 128 // d
    farange = jnp.arange(f, dtype=jnp.int32)
    offsets = ((farange // fold) * (v * fold) + (farange % fold))[None, :]
    flat_idx = (indices.astype(jnp.int32).T * fold + offsets).reshape(
        nw, rpw // CHUNK, CHUNK
    )
    packed = _tc_repack(jnp.transpose(tables, (0, 2, 1)), f, v, d)
    flat_tables = packed.reshape(packed.shape[0] * v * fold, d)

    rows = _sc_gather(flat_idx, flat_tables, nw, rpw, d)
    combined = rows.reshape(b, f * d)
    return _tc_mlp(combined, W1, b1, W2, b2)


# speedup vs baseline: 18.8770x; 1.0027x over previous
import functools

import jax
import jax.numpy as jnp
from jax import lax
from jax.experimental import pallas as pl
from jax.experimental.pallas import tpu as pltpu
from jax.experimental.pallas import tpu_sc as plsc

CHUNK = 128
GROUP = 8
VCHUNK = 12800
LANES = 128


def _tc_repack(t_dv, g, v, d, nf):
    fold = LANES // d
    start = g * fold
    nv = (v + VCHUNK - 1) // VCHUNK

    def body(x_ref, o_ref):
        acc = None
        for k in range(nf):
            e_k = (
                lax.broadcasted_iota(jnp.int32, (d, LANES), 0) + k * d
                == lax.broadcasted_iota(jnp.int32, (d, LANES), 1)
            ).astype(jnp.float32)
            p = lax.dot_general(
                x_ref[k],
                e_k,
                (((0,), (0,)), ((), ())),
                preferred_element_type=jnp.float32,
            )
            acc = p if acc is None else acc + p
        o_ref[...] = acc

    return pl.pallas_call(
        body,
        grid=(nv,),
        in_specs=[pl.BlockSpec((nf, d, VCHUNK), lambda j: (start // nf, 0, j))],
        out_specs=pl.BlockSpec((VCHUNK, LANES), lambda j: (j, 0)),
        out_shape=jax.ShapeDtypeStruct((v, LANES), jnp.float32),
    )(t_dv)


def _sc_gather(flat_idx, flat_tables, nw, rpw, d):
    nch = rpw // CHUNK
    ngroups = nch // GROUP
    grows = GROUP * CHUNK
    mesh = plsc.VectorSubcoreMesh(core_axis_name="c", subcore_axis_name="s")
    nc = 2

    @functools.partial(
        pl.kernel,
        mesh=mesh,
        compiler_params=pltpu.CompilerParams(use_tc_tiling_on_sc=False),
        out_type=jax.ShapeDtypeStruct((nw, rpw, d), jnp.float32),
        scratch_types=[
            pltpu.VMEM((nch, CHUNK), jnp.int32),
            pltpu.VMEM((grows, d), jnp.float32),
            pltpu.SemaphoreType.DMA,
        ],
    )
    def k(idx_hbm, tab_hbm, out_hbm, idx_v, rows_v, sem):
        wid = lax.axis_index("s") * nc + lax.axis_index("c")
        pltpu.sync_copy(idx_hbm.at[wid], idx_v)

        def body(gg, carry):
            handles = []
            for j in range(GROUP):
                h = pltpu.async_copy(
                    tab_hbm.at[idx_v.at[gg * GROUP + j]],
                    rows_v.at[pl.ds(j * CHUNK, CHUNK)],
                    sem,
                )
                handles.append(h)
            for h in handles:
                h.wait()
            pltpu.sync_copy(rows_v, out_hbm.at[wid, pl.ds(gg * grows, grows)])
            return carry

        lax.fori_loop(0, ngroups, body, 0)

    return k(flat_idx, flat_tables)


def _tc_mlp(xs, w1s, b1, W2, b2, tb=1024):
    bsz = xs[0].shape[0]
    h1 = w1s[0].shape[1]
    ed = W2.shape[1]
    ng = len(xs)

    def body(*refs):
        x_refs = refs[:ng]
        w_refs = refs[ng:2 * ng]
        b1_ref, w2_ref, b2_ref, o_ref = refs[2 * ng:]
        h = b1_ref[...]
        for xr, wr in zip(x_refs, w_refs):
            h = h + jnp.dot(xr[...], wr[...], preferred_element_type=jnp.float32)
        h = jnp.maximum(h, 0.0)
        o = jnp.dot(h, w2_ref[...], preferred_element_type=jnp.float32)
        o_ref[...] = jnp.maximum(o + b2_ref[...], 0.0)

    in_specs = (
        [pl.BlockSpec((tb, x.shape[1]), lambda i: (i, 0)) for x in xs]
        + [pl.BlockSpec(w.shape, lambda i: (0, 0)) for w in w1s]
        + [
            pl.BlockSpec((1, h1), lambda i: (0, 0)),
            pl.BlockSpec(W2.shape, lambda i: (0, 0)),
            pl.BlockSpec((1, ed), lambda i: (0, 0)),
        ]
    )
    return pl.pallas_call(
        body,
        grid=(bsz // tb,),
        in_specs=in_specs,
        out_specs=pl.BlockSpec((tb, ed), lambda i: (i, 0)),
        out_shape=jax.ShapeDtypeStruct((bsz, ed), jnp.float32),
    )(*xs, *w1s, b1.reshape(1, h1), W2, b2.reshape(1, ed))


def kernel(indices, tables, W1, b1, W2, b2):
    f, b = indices.shape
    _, v, d = tables.shape
    nw = 32
    fold = LANES // d

    t_dv = jnp.transpose(tables, (0, 2, 1))
    idx32 = indices.astype(jnp.int32)

    ngrp = (f + fold - 1) // fold
    xs, w1s = [], []
    for g in range(ngrp):
        nf = min(fold, f - g * fold)
        packed = _tc_repack(t_dv, g, v, d, nf)
        sub = idx32[g * fold : g * fold + nf].T
        flat = sub * fold + jnp.arange(nf, dtype=jnp.int32)[None, :]
        rpw = b * nf // nw
        flat_idx = flat.reshape(nw, rpw // CHUNK, CHUNK)
        rows = _sc_gather(flat_idx, packed.reshape(v * fold, d), nw, rpw, d)
        xs.append(rows.reshape(b, nf * d))
        w1s.append(W1[g * fold * d : (g * fold + nf) * d])

    return _tc_mlp(xs, w1s, b1, W2, b2)
